# Initial kernel scaffold; baseline (speedup 1.0000x reference)
import jax
import jax.numpy as jnp
from jax.experimental import pallas as pl


def kernel(x, edge_index, style, W_edge, b_edge, Wc1, bc1, Wc2, bc2, l0W, l0b, a0W, a0b, ns0, nz0, l1W, l1b, a1W, a1b, ns1, nz1, l2W, l2b, a2W, a2b, ns2, nz2):
    raise NotImplementedError("write your pallas kernel here")


# speedup vs baseline: 6.4868x; 6.4868x over previous
import functools

import jax
import jax.numpy as jnp
from jax import lax
from jax.experimental import pallas as pl
from jax.experimental.pallas import tpu as pltpu
from jax.experimental.pallas import tpu_sc as plsc

_N = 10000
_C = 128
_E = 320000
_EPS = 1e-5

_NC = 2
_NS = 16
_NW = _NC * _NS
_EPW = _E // _NW
_K = 128
_FULL = _EPW // _K
_REM = _EPW - _FULL * _K
_RPT = 624
_TAIL = _N - _RPT * _NS

_mesh = plsc.VectorSubcoreMesh(core_axis_name="c", subcore_axis_name="s")


@functools.partial(
    pl.kernel,
    out_type=jax.ShapeDtypeStruct((_NC, _N, _C), jnp.float32),
    mesh=_mesh,
    scratch_types=[
        pltpu.VMEM((_K,), jnp.int32),
        pltpu.VMEM((_K,), jnp.int32),
        pltpu.VMEM((_K, _C), jnp.float32),
        pltpu.VMEM((_REM,), jnp.int32),
        pltpu.VMEM((_REM,), jnp.int32),
        pltpu.VMEM((_REM, _C), jnp.float32),
        pltpu.VMEM_SHARED((_N, _C), jnp.float32),
        pltpu.SemaphoreType.DMA,
    ],
)
def _sc_segment_sum(srce_ref, dste_ref, wedge_ref, zeros_ref, out_ref,
                    src_v, dst_v, rows_v, src_r, dst_r, rows_r, acc_sh, sem):
    cid = lax.axis_index("c")
    sid = lax.axis_index("s")
    wid = cid * _NS + sid
    r0 = sid * _RPT
    pltpu.sync_copy(zeros_ref.at[pl.ds(r0, _RPT)], acc_sh.at[pl.ds(r0, _RPT)])

    @pl.when(sid == _NS - 1)
    def _():
        t0 = _RPT * _NS
        pltpu.sync_copy(zeros_ref.at[pl.ds(t0, _TAIL)],
                        acc_sh.at[pl.ds(t0, _TAIL)])

    plsc.subcore_barrier()

    e0 = wid * _EPW

    def chunk(j, carry):
        base = e0 + j * _K
        pltpu.sync_copy(srce_ref.at[pl.ds(base, _K)], src_v)
        pltpu.sync_copy(dste_ref.at[pl.ds(base, _K)], dst_v)
        pltpu.async_copy(wedge_ref.at[src_v], rows_v, sem).wait()
        pltpu.sync_copy(rows_v, acc_sh.at[dst_v], add=True)
        return carry

    lax.fori_loop(0, _FULL, chunk, 0)

    rbase = e0 + _FULL * _K
    pltpu.sync_copy(srce_ref.at[pl.ds(rbase, _REM)], src_r)
    pltpu.sync_copy(dste_ref.at[pl.ds(rbase, _REM)], dst_r)
    pltpu.async_copy(wedge_ref.at[src_r], rows_r, sem).wait()
    pltpu.sync_copy(rows_r, acc_sh.at[dst_r], add=True)

    plsc.subcore_barrier()
    pltpu.sync_copy(acc_sh.at[pl.ds(r0, _RPT)],
                    out_ref.at[cid, pl.ds(r0, _RPT)])

    @pl.when(sid == _NS - 1)
    def _():
        t0 = _RPT * _NS
        pltpu.sync_copy(acc_sh.at[pl.ds(t0, _TAIL)],
                        out_ref.at[cid, pl.ds(t0, _TAIL)])


def _mm(a, b):
    return lax.dot_general(a, b, (((1,), (1,)), ((), ())),
                           preferred_element_type=jnp.float32)


def _style(h_in, sty, lW, lb, aWg, aWb, abg, abb, nzs):
    h = _mm(h_in, lW) + lb + nzs
    gamma = _mm(sty, aWg) + abg
    beta = _mm(sty, aWb) + abb
    mu = jnp.mean(h, axis=0, keepdims=True)
    var = jnp.mean((h - mu) * (h - mu), axis=0, keepdims=True)
    hn = (h - mu) * lax.rsqrt(var + _EPS)
    h = gamma * hn + beta
    return jnp.where(h >= 0, h, 0.01 * h)


def _tc_body(acc_ref, x_ref, style_ref, bedge_ref, Wc1_ref, bc1_ref,
             Wc2_ref, bc2_ref,
             l0W_ref, l0b_ref, a0Wg_ref, a0Wb_ref, a0bg_ref, a0bb_ref, nzs0_ref,
             l1W_ref, l1b_ref, a1Wg_ref, a1Wb_ref, a1bg_ref, a1bb_ref, nzs1_ref,
             l2W_ref, l2b_ref, a2Wg_ref, a2Wb_ref, a2bg_ref, a2bb_ref, nzs2_ref,
             out_ref):
    x = x_ref[...]
    sty = style_ref[...]
    out = acc_ref[0] + acc_ref[1] + bedge_ref[...]
    out = out + _mm(out, Wc1_ref[...]) + bc1_ref[...]
    xm = _style(x, sty, l0W_ref[...], l0b_ref[...], a0Wg_ref[...],
                a0Wb_ref[...], a0bg_ref[...], a0bb_ref[...], nzs0_ref[...])
    out = out + xm
    out = out + _mm(xm, Wc2_ref[...]) + bc2_ref[...]
    out = jnp.maximum(out, 0.0)
    out = _style(out, sty, l1W_ref[...], l1b_ref[...], a1Wg_ref[...],
                 a1Wb_ref[...], a1bg_ref[...], a1bb_ref[...], nzs1_ref[...])
    out = _style(out, sty, l2W_ref[...], l2b_ref[...], a2Wg_ref[...],
                 a2Wb_ref[...], a2bg_ref[...], a2bb_ref[...], nzs2_ref[...])
    out_ref[...] = out


_tc_call = pl.pallas_call(
    _tc_body,
    out_shape=jax.ShapeDtypeStruct((_N, _C), jnp.float32),
)


def kernel(x, edge_index, style, W_edge, b_edge, Wc1, bc1, Wc2, bc2,
           l0W, l0b, a0W, a0b, ns0, nz0,
           l1W, l1b, a1W, a1b, ns1, nz1,
           l2W, l2b, a2W, a2b, ns2, nz2):
    ei = edge_index.astype(jnp.int32)
    zeros = jnp.zeros((_N, _C), jnp.float32)
    acc = _sc_segment_sum(ei[0], ei[1], W_edge, zeros)

    def prep(aW, ab, ns, nz):
        return (aW[:_C], aW[_C:], ab[:_C].reshape(1, _C),
                ab[_C:].reshape(1, _C), (ns * nz).reshape(1, _C))

    a0Wg, a0Wb, a0bg, a0bb, nzs0 = prep(a0W, a0b, ns0, nz0)
    a1Wg, a1Wb, a1bg, a1bb, nzs1 = prep(a1W, a1b, ns1, nz1)
    a2Wg, a2Wb, a2bg, a2bb, nzs2 = prep(a2W, a2b, ns2, nz2)

    return _tc_call(
        acc, x, style, b_edge.reshape(1, _C), Wc1, bc1.reshape(1, _C),
        Wc2, bc2.reshape(1, _C),
        l0W, l0b.reshape(1, _C), a0Wg, a0Wb, a0bg, a0bb, nzs0,
        l1W, l1b.reshape(1, _C), a1Wg, a1Wb, a1bg, a1bb, nzs1,
        l2W, l2b.reshape(1, _C), a2Wg, a2Wb, a2bg, a2bb, nzs2)
